# baseline (speedup 1.0000x reference)
import functools

import jax
import jax.numpy as jnp
from jax import lax
from jax.experimental import pallas as pl
from jax.experimental.pallas import tpu as pltpu
from jax.experimental.pallas import tpu_sc as plsc

VOCAB = 1000000
EMBED = 64
BATCH = 4096
SEQ = 200
TOTAL = BATCH * SEQ

NUM_CORES = 2
NUM_SUBCORES = 16
NW = NUM_CORES * NUM_SUBCORES
PER_W = TOTAL // NW
CHUNK = 128
ROWS_W = PER_W // CHUNK
K = 4
GROUP = K * CHUNK
N_GROUPS = PER_W // GROUP


def _gather_body(idx_hbm, table_hbm, out_hbm, idx_v, rows_v, sem_g):
    c = lax.axis_index("c")
    s = lax.axis_index("s")
    wid = s * NUM_CORES + c
    pltpu.sync_copy(idx_hbm.at[wid], idx_v)
    out_base = wid * PER_W

    def group(g, carry):
        cps = []
        for j in range(K):
            cp = pltpu.async_copy(
                table_hbm.at[idx_v.at[g * K + j]],
                rows_v.at[pl.ds(j * CHUNK, CHUNK)],
                sem_g,
            )
            cps.append(cp)
        for cp in cps:
            cp.wait()
        pltpu.sync_copy(rows_v, out_hbm.at[pl.ds(out_base + g * GROUP, GROUP)])
        return carry

    lax.fori_loop(0, N_GROUPS, group, 0)


@functools.partial(jax.jit, static_argnames=())
def _embedding_gather(idx2d, table):
    mesh = plsc.VectorSubcoreMesh(core_axis_name="c", subcore_axis_name="s")
    kern = functools.partial(
        pl.kernel,
        mesh=mesh,
        out_type=jax.ShapeDtypeStruct((TOTAL, EMBED), jnp.float32),
        scratch_types=[
            pltpu.VMEM((ROWS_W, CHUNK), jnp.int32),
            pltpu.VMEM((GROUP, EMBED), jnp.float32),
            pltpu.SemaphoreType.DMA,
        ],
        compiler_params=pltpu.CompilerParams(use_tc_tiling_on_sc=False),
    )(_gather_body)
    return kern(idx2d, table)


def kernel(input_sequence, word_embed_weight):
    idx2d = input_sequence.astype(jnp.int32).reshape(NW, ROWS_W, CHUNK)
    out = _embedding_gather(idx2d, word_embed_weight)
    return out.reshape(BATCH, SEQ, EMBED)


# speedup vs baseline: 1.0242x; 1.0242x over previous
import functools

import jax
import jax.numpy as jnp
from jax import lax
from jax.experimental import pallas as pl
from jax.experimental.pallas import tpu as pltpu
from jax.experimental.pallas import tpu_sc as plsc

VOCAB = 1000000
EMBED = 64
BATCH = 4096
SEQ = 200
TOTAL = BATCH * SEQ

NUM_CORES = 2
NUM_SUBCORES = 16
NW = NUM_CORES * NUM_SUBCORES
PER_W = TOTAL // NW
CHUNK = 128
ROWS_W = PER_W // CHUNK
K = 4
GROUP = K * CHUNK
N_GROUPS = PER_W // GROUP


NBUF = 3


def _gather_body(idx_hbm, table_hbm, out_hbm, idx_v, rows_v, sem_g, sem_o):
    c = lax.axis_index("c")
    s = lax.axis_index("s")
    wid = s * NUM_CORES + c
    pltpu.sync_copy(idx_hbm.at[wid], idx_v)
    out_base = wid * PER_W

    def fire(g, slot):
        for j in range(K):
            pltpu.make_async_copy(
                table_hbm.at[idx_v.at[g * K + j]],
                rows_v.at[slot, pl.ds(j * CHUNK, CHUNK)],
                sem_g.at[slot],
            ).start()

    def drain(g, slot):
        for j in range(K):
            pltpu.make_async_copy(
                table_hbm.at[idx_v.at[g * K + j]],
                rows_v.at[slot, pl.ds(j * CHUNK, CHUNK)],
                sem_g.at[slot],
            ).wait()

    def store_desc(g, slot):
        return pltpu.make_async_copy(
            rows_v.at[slot],
            out_hbm.at[pl.ds(out_base + g * GROUP, GROUP)],
            sem_o.at[slot],
        )

    fire(0, 0)
    fire(1, 1)

    def step(g, carry):
        slot = lax.rem(g, NBUF)
        drain(g, slot)
        store_desc(g, slot).start()
        slot_next = lax.rem(g + 2, NBUF)

        @pl.when(g >= 1)
        def _():
            store_desc(g - 1, slot_next).wait()

        @pl.when(g + 2 < N_GROUPS)
        def _():
            fire(g + 2, slot_next)

        return carry

    lax.fori_loop(0, N_GROUPS, step, 0)
    store_desc(N_GROUPS - 1, lax.rem(N_GROUPS - 1, NBUF)).wait()


@functools.partial(jax.jit, static_argnames=())
def _embedding_gather(idx2d, table):
    mesh = plsc.VectorSubcoreMesh(core_axis_name="c", subcore_axis_name="s")
    kern = functools.partial(
        pl.kernel,
        mesh=mesh,
        out_type=jax.ShapeDtypeStruct((TOTAL, EMBED), jnp.float32),
        scratch_types=[
            pltpu.VMEM((ROWS_W, CHUNK), jnp.int32),
            pltpu.VMEM((NBUF, GROUP, EMBED), jnp.float32),
            pltpu.SemaphoreType.DMA((NBUF,)),
            pltpu.SemaphoreType.DMA((NBUF,)),
        ],
        compiler_params=pltpu.CompilerParams(use_tc_tiling_on_sc=False),
    )(_gather_body)
    return kern(idx2d, table)


def kernel(input_sequence, word_embed_weight):
    idx2d = input_sequence.astype(jnp.int32).reshape(NW, ROWS_W, CHUNK)
    out = _embedding_gather(idx2d, word_embed_weight)
    return out.reshape(BATCH, SEQ, EMBED)


# speedup vs baseline: 1.0251x; 1.0009x over previous
import functools

import jax
import jax.numpy as jnp
from jax import lax
from jax.experimental import pallas as pl
from jax.experimental.pallas import tpu as pltpu
from jax.experimental.pallas import tpu_sc as plsc

VOCAB = 1000000
EMBED = 64
BATCH = 4096
SEQ = 200
TOTAL = BATCH * SEQ

NUM_CORES = 2
NUM_SUBCORES = 16
NW = NUM_CORES * NUM_SUBCORES
PER_W = TOTAL // NW
CHUNK = 512
ROWS_W = PER_W // CHUNK
K = 1
GROUP = K * CHUNK
N_GROUPS = PER_W // GROUP


NBUF = 3


def _gather_body(idx_hbm, table_hbm, out_hbm, idx_v, rows_v, sem_g, sem_o):
    c = lax.axis_index("c")
    s = lax.axis_index("s")
    wid = s * NUM_CORES + c
    pltpu.sync_copy(idx_hbm.at[wid], idx_v)
    out_base = wid * PER_W

    def fire(g, slot):
        for j in range(K):
            pltpu.make_async_copy(
                table_hbm.at[idx_v.at[g * K + j]],
                rows_v.at[slot, pl.ds(j * CHUNK, CHUNK)],
                sem_g.at[slot],
            ).start()

    def drain(g, slot):
        for j in range(K):
            pltpu.make_async_copy(
                table_hbm.at[idx_v.at[g * K + j]],
                rows_v.at[slot, pl.ds(j * CHUNK, CHUNK)],
                sem_g.at[slot],
            ).wait()

    def store_desc(g, slot):
        return pltpu.make_async_copy(
            rows_v.at[slot],
            out_hbm.at[pl.ds(out_base + g * GROUP, GROUP)],
            sem_o.at[slot],
        )

    fire(0, 0)
    fire(1, 1)

    def step(g, carry):
        slot = lax.rem(g, NBUF)
        drain(g, slot)
        store_desc(g, slot).start()
        slot_next = lax.rem(g + 2, NBUF)

        @pl.when(g >= 1)
        def _():
            store_desc(g - 1, slot_next).wait()

        @pl.when(g + 2 < N_GROUPS)
        def _():
            fire(g + 2, slot_next)

        return carry

    lax.fori_loop(0, N_GROUPS, step, 0)
    store_desc(N_GROUPS - 1, lax.rem(N_GROUPS - 1, NBUF)).wait()


@functools.partial(jax.jit, static_argnames=())
def _embedding_gather(idx2d, table):
    mesh = plsc.VectorSubcoreMesh(core_axis_name="c", subcore_axis_name="s")
    kern = functools.partial(
        pl.kernel,
        mesh=mesh,
        out_type=jax.ShapeDtypeStruct((TOTAL, EMBED), jnp.float32),
        scratch_types=[
            pltpu.VMEM((ROWS_W, CHUNK), jnp.int32),
            pltpu.VMEM((NBUF, GROUP, EMBED), jnp.float32),
            pltpu.SemaphoreType.DMA((NBUF,)),
            pltpu.SemaphoreType.DMA((NBUF,)),
        ],
        compiler_params=pltpu.CompilerParams(use_tc_tiling_on_sc=False),
    )(_gather_body)
    return kern(idx2d, table)


def kernel(input_sequence, word_embed_weight):
    idx2d = input_sequence.astype(jnp.int32).reshape(NW, ROWS_W, CHUNK)
    out = _embedding_gather(idx2d, word_embed_weight)
    return out.reshape(BATCH, SEQ, EMBED)


# speedup vs baseline: 1.4322x; 1.3971x over previous
import functools

import jax
import jax.numpy as jnp
from jax import lax
from jax.experimental import pallas as pl
from jax.experimental.pallas import tpu as pltpu
from jax.experimental.pallas import tpu_sc as plsc

VOCAB = 1000000
EMBED = 64
BATCH = 4096
SEQ = 200


TCH = 2048
NCH = (VOCAB + TCH - 1) // TCH
TAIL = VOCAB - (NCH - 1) * TCH
HALF = TCH // 2
TBL2_ROWS = NCH * HALF
TAIL_A = TAIL - (TAIL % 128)
TAIL_B = TAIL % 128


def _tbl_relayout_body(x_hbm, o_ref, scr, scr_tail, sem, sem_t):
    g = pl.program_id(0)

    def descs(i, slot):
        full = pltpu.make_async_copy(
            x_hbm.at[:, pl.ds(i * TCH, TCH)], scr.at[slot], sem.at[slot]
        )
        part = pltpu.make_async_copy(
            x_hbm.at[:, pl.ds((NCH - 1) * TCH, TAIL_A)],
            scr.at[slot, :, pl.ds(0, TAIL_A)],
            sem.at[slot],
        )
        tail = pltpu.make_async_copy(
            x_hbm.at[:, pl.ds((NCH - 1) * TCH + TAIL_A, TAIL_B)], scr_tail, sem_t
        )
        return full, part, tail

    def dma(i, slot):
        full, part, tail = descs(i, slot)

        @pl.when(i < NCH - 1)
        def _():
            full.start()

        @pl.when(i == NCH - 1)
        def _():
            part.start()
            tail.start()

    def dma_wait(i, slot):
        full, part, tail = descs(i, slot)

        @pl.when(i < NCH - 1)
        def _():
            full.wait()

        @pl.when(i == NCH - 1)
        def _():
            part.wait()
            tail.wait()

    @pl.when(g == 0)
    def _():
        dma(0, 0)

    @pl.when(g + 1 < NCH)
    def _():
        dma(g + 1, (g + 1) % 2)

    dma_wait(g, g % 2)

    @pl.when(g == NCH - 1)
    def _():
        scr[g % 2, :, TAIL_A:TAIL] = scr_tail[...]

    x = scr[g % 2]
    o_ref[:, 0:EMBED] = x[:, 0:HALF].T
    o_ref[:, EMBED : 2 * EMBED] = x[:, HALF:TCH].T


def _relayout_table(table_t):
    return pl.pallas_call(
        _tbl_relayout_body,
        grid=(NCH,),
        in_specs=[pl.BlockSpec(memory_space=pltpu.MemorySpace.HBM)],
        out_specs=pl.BlockSpec((HALF, 2 * EMBED), lambda g: (g, 0)),
        out_shape=jax.ShapeDtypeStruct((TBL2_ROWS, 2 * EMBED), jnp.float32),
        scratch_shapes=[
            pltpu.VMEM((2, EMBED, TCH), jnp.float32),
            pltpu.VMEM((EMBED, TAIL_B), jnp.float32),
            pltpu.SemaphoreType.DMA((2,)),
            pltpu.SemaphoreType.DMA,
        ],
    )(table_t)


NUM_CORES = 2
NUM_SUBCORES = 16
NW = NUM_CORES * NUM_SUBCORES
COLS_W = BATCH // NW
GL = 4
N_GROUPS = SEQ // GL
NBUF = 3


def _gather_body(idx_hbm, table_hbm, out_hbm, idx_v, rows_v, sem_g, sem_o):
    c = lax.axis_index("c")
    s = lax.axis_index("s")
    wid = s * NUM_CORES + c
    c0 = wid * COLS_W
    pltpu.sync_copy(idx_hbm.at[:, pl.ds(c0, COLS_W)], idx_v)

    def fire(g, slot):
        for j in range(GL):
            pltpu.make_async_copy(
                table_hbm.at[idx_v.at[g * GL + j]],
                rows_v.at[slot, j],
                sem_g.at[slot],
            ).start()

    def drain(g, slot):
        for j in range(GL):
            pltpu.make_async_copy(
                table_hbm.at[idx_v.at[g * GL + j]],
                rows_v.at[slot, j],
                sem_g.at[slot],
            ).wait()

    def store_desc(g, slot):
        return pltpu.make_async_copy(
            rows_v.at[slot],
            out_hbm.at[pl.ds(g * GL, GL), pl.ds(c0, COLS_W)],
            sem_o.at[slot],
        )

    fire(0, 0)
    fire(1, 1)

    def step(g, carry):
        slot = lax.rem(g, NBUF)
        drain(g, slot)
        store_desc(g, slot).start()
        slot_next = lax.rem(g + 2, NBUF)

        @pl.when(g >= 1)
        def _():
            store_desc(g - 1, slot_next).wait()

        @pl.when(g + 2 < N_GROUPS)
        def _():
            fire(g + 2, slot_next)

        return carry

    lax.fori_loop(0, N_GROUPS, step, 0)
    store_desc(N_GROUPS - 1, lax.rem(N_GROUPS - 1, NBUF)).wait()


def _embedding_gather(idx_p, table_lin):
    mesh = plsc.VectorSubcoreMesh(core_axis_name="c", subcore_axis_name="s")
    kern = functools.partial(
        pl.kernel,
        mesh=mesh,
        out_type=jax.ShapeDtypeStruct((SEQ, BATCH, EMBED), jnp.float32),
        scratch_types=[
            pltpu.VMEM((SEQ, COLS_W), jnp.int32),
            pltpu.VMEM((NBUF, GL, COLS_W, EMBED), jnp.float32),
            pltpu.SemaphoreType.DMA((NBUF,)),
            pltpu.SemaphoreType.DMA((NBUF,)),
        ],
        compiler_params=pltpu.CompilerParams(use_tc_tiling_on_sc=False),
    )(_gather_body)
    return kern(idx_p, table_lin)


HB = BATCH // 2


def _out_relayout_body(x_ref, o_ref):
    x = x_ref[...]
    o_ref[0, :, 0:HB] = x[:, 0:EMBED].T
    o_ref[0, :, HB:BATCH] = x[:, EMBED : 2 * EMBED].T


def _relayout_out(out2_rows):
    return pl.pallas_call(
        _out_relayout_body,
        grid=(SEQ,),
        in_specs=[pl.BlockSpec((HB, 2 * EMBED), lambda l: (l, 0))],
        out_specs=pl.BlockSpec((1, EMBED, BATCH), lambda l: (l, 0, 0)),
        out_shape=jax.ShapeDtypeStruct((SEQ, EMBED, BATCH), jnp.float32),
    )(out2_rows)


@jax.jit
def _run(input_sequence, word_embed_weight):
    tbl2 = _relayout_table(word_embed_weight.T)
    tbl_lin = tbl2.reshape(2 * TBL2_ROWS, EMBED)

    v = input_sequence.astype(jnp.int32)
    q = ((v >> 11) << 11) + ((v & 1023) << 1) + ((v >> 10) & 1)
    qt = q.T
    idx_p = jnp.stack((qt[:, :HB], qt[:, HB:]), axis=2).reshape(SEQ, BATCH)

    out2 = _embedding_gather(idx_p, tbl_lin)
    out_t = _relayout_out(out2.reshape(SEQ * HB, 2 * EMBED))
    return out_t.transpose(2, 0, 1)


def kernel(input_sequence, word_embed_weight):
    return _run(input_sequence, word_embed_weight)


# speedup vs baseline: 1.6674x; 1.1642x over previous
import functools

import jax
import jax.numpy as jnp
from jax import lax
from jax.experimental import pallas as pl
from jax.experimental.pallas import tpu as pltpu
from jax.experimental.pallas import tpu_sc as plsc

VOCAB = 1000000
EMBED = 64
BATCH = 4096
SEQ = 200


TCH = 2048
NCH = (VOCAB + TCH - 1) // TCH
TAIL = VOCAB - (NCH - 1) * TCH
HALF = TCH // 2
TBL2_ROWS = NCH * HALF
TAIL_A = TAIL - (TAIL % 128)
TAIL_B = TAIL % 128
TNB = 4


def _tbl_relayout_body(x_hbm, o_ref, scr, scr_tail, sem, sem_t):
    g = pl.program_id(0)

    def descs(i, slot):
        full = pltpu.make_async_copy(
            x_hbm.at[:, pl.ds(i * TCH, TCH)], scr.at[slot], sem.at[slot]
        )
        part = pltpu.make_async_copy(
            x_hbm.at[:, pl.ds((NCH - 1) * TCH, TAIL_A)],
            scr.at[slot, :, pl.ds(0, TAIL_A)],
            sem.at[slot],
        )
        tail = pltpu.make_async_copy(
            x_hbm.at[:, pl.ds((NCH - 1) * TCH + TAIL_A, TAIL_B)], scr_tail, sem_t
        )
        return full, part, tail

    def dma(i, slot):
        full, part, tail = descs(i, slot)

        @pl.when(i < NCH - 1)
        def _():
            full.start()

        @pl.when(i == NCH - 1)
        def _():
            part.start()
            tail.start()

    def dma_wait(i, slot):
        full, part, tail = descs(i, slot)

        @pl.when(i < NCH - 1)
        def _():
            full.wait()

        @pl.when(i == NCH - 1)
        def _():
            part.wait()
            tail.wait()

    @pl.when(g == 0)
    def _():
        for i in range(TNB - 1):
            dma(i, i)

    @pl.when(g + TNB - 1 < NCH)
    def _():
        dma(g + TNB - 1, lax.rem(g + TNB - 1, TNB))

    dma_wait(g, lax.rem(g, TNB))

    @pl.when(g == NCH - 1)
    def _():
        scr[lax.rem(g, TNB), :, TAIL_A:TAIL] = scr_tail[...]

    x = scr[lax.rem(g, TNB)]
    o_ref[:, 0:EMBED] = x[:, 0:HALF].T
    o_ref[:, EMBED : 2 * EMBED] = x[:, HALF:TCH].T


def _relayout_table(table_t):
    return pl.pallas_call(
        _tbl_relayout_body,
        grid=(NCH,),
        in_specs=[pl.BlockSpec(memory_space=pltpu.MemorySpace.HBM)],
        out_specs=pl.BlockSpec((HALF, 2 * EMBED), lambda g: (g, 0)),
        out_shape=jax.ShapeDtypeStruct((TBL2_ROWS, 2 * EMBED), jnp.float32),
        scratch_shapes=[
            pltpu.VMEM((TNB, EMBED, TCH), jnp.float32),
            pltpu.VMEM((EMBED, TAIL_B), jnp.float32),
            pltpu.SemaphoreType.DMA((TNB,)),
            pltpu.SemaphoreType.DMA,
        ],
    )(table_t)


NUM_CORES = 2
NUM_SUBCORES = 16
NW = NUM_CORES * NUM_SUBCORES
COLS_W = BATCH // NW
GL = 4
N_GROUPS = SEQ // GL
NBUF = 3


def _gather_body(idx_hbm, table_hbm, out_hbm, idx_v, rows_v, sem_g, sem_o):
    c = lax.axis_index("c")
    s = lax.axis_index("s")
    wid = s * NUM_CORES + c
    c0 = wid * COLS_W
    pltpu.sync_copy(idx_hbm.at[:, pl.ds(c0, COLS_W)], idx_v)

    def fire(g, slot):
        for j in range(GL):
            pltpu.make_async_copy(
                table_hbm.at[idx_v.at[g * GL + j]],
                rows_v.at[slot, j],
                sem_g.at[slot],
            ).start()

    def drain(g, slot):
        for j in range(GL):
            pltpu.make_async_copy(
                table_hbm.at[idx_v.at[g * GL + j]],
                rows_v.at[slot, j],
                sem_g.at[slot],
            ).wait()

    def store_desc(g, slot):
        return pltpu.make_async_copy(
            rows_v.at[slot],
            out_hbm.at[pl.ds(g * GL, GL), pl.ds(c0, COLS_W)],
            sem_o.at[slot],
        )

    fire(0, 0)
    fire(1, 1)

    def step(g, carry):
        slot = lax.rem(g, NBUF)
        drain(g, slot)
        store_desc(g, slot).start()
        slot_next = lax.rem(g + 2, NBUF)

        @pl.when(g >= 1)
        def _():
            store_desc(g - 1, slot_next).wait()

        @pl.when(g + 2 < N_GROUPS)
        def _():
            fire(g + 2, slot_next)

        return carry

    lax.fori_loop(0, N_GROUPS, step, 0)
    store_desc(N_GROUPS - 1, lax.rem(N_GROUPS - 1, NBUF)).wait()


def _embedding_gather(idx_p, table_lin):
    mesh = plsc.VectorSubcoreMesh(core_axis_name="c", subcore_axis_name="s")
    kern = functools.partial(
        pl.kernel,
        mesh=mesh,
        out_type=jax.ShapeDtypeStruct((SEQ, BATCH, EMBED), jnp.float32),
        scratch_types=[
            pltpu.VMEM((SEQ, COLS_W), jnp.int32),
            pltpu.VMEM((NBUF, GL, COLS_W, EMBED), jnp.float32),
            pltpu.SemaphoreType.DMA((NBUF,)),
            pltpu.SemaphoreType.DMA((NBUF,)),
        ],
        compiler_params=pltpu.CompilerParams(use_tc_tiling_on_sc=False),
    )(_gather_body)
    return kern(idx_p, table_lin)


HB = BATCH // 2


def _out_relayout_body(x_ref, o_ref):
    x = x_ref[...]
    o_ref[0, :, 0:HB] = x[:, 0:EMBED].T
    o_ref[0, :, HB:BATCH] = x[:, EMBED : 2 * EMBED].T


def _relayout_out(out2_rows):
    return pl.pallas_call(
        _out_relayout_body,
        grid=(SEQ,),
        in_specs=[pl.BlockSpec((HB, 2 * EMBED), lambda l: (l, 0))],
        out_specs=pl.BlockSpec((1, EMBED, BATCH), lambda l: (l, 0, 0)),
        out_shape=jax.ShapeDtypeStruct((SEQ, EMBED, BATCH), jnp.float32),
    )(out2_rows)


@jax.jit
def _run(input_sequence, word_embed_weight):
    tbl2 = _relayout_table(word_embed_weight.T)
    tbl_lin = tbl2.reshape(2 * TBL2_ROWS, EMBED)

    v = input_sequence.astype(jnp.int32)
    q = ((v >> 11) << 11) + ((v & 1023) << 1) + ((v >> 10) & 1)
    qt = q.T
    idx_p = jnp.stack((qt[:, :HB], qt[:, HB:]), axis=2).reshape(SEQ, BATCH)

    out2 = _embedding_gather(idx_p, tbl_lin)
    out_t = _relayout_out(out2.reshape(SEQ * HB, 2 * EMBED))
    return out_t.transpose(2, 0, 1)


def kernel(input_sequence, word_embed_weight):
    return _run(input_sequence, word_embed_weight)


# speedup vs baseline: 2.0238x; 1.2137x over previous
import functools

import jax
import jax.numpy as jnp
from jax import lax
from jax.experimental import pallas as pl
from jax.experimental.pallas import tpu as pltpu
from jax.experimental.pallas import tpu_sc as plsc

VOCAB = 1000000
EMBED = 64
BATCH = 4096
SEQ = 200


TCH = 4096
NCH = (VOCAB + TCH - 1) // TCH
TAIL = VOCAB - (NCH - 1) * TCH
HALF = TCH // 2
TBL2_ROWS = NCH * HALF
TAIL_A = TAIL - (TAIL % 128)
TAIL_B = TAIL % 128
TNB = 4


def _tbl_relayout_body(x_hbm, o_ref, scr, scr_tail, sem, sem_t):
    g = pl.program_id(0)

    def descs(i, slot):
        full = pltpu.make_async_copy(
            x_hbm.at[:, pl.ds(i * TCH, TCH)], scr.at[slot], sem.at[slot]
        )
        part = pltpu.make_async_copy(
            x_hbm.at[:, pl.ds((NCH - 1) * TCH, TAIL_A)],
            scr.at[slot, :, pl.ds(0, TAIL_A)],
            sem.at[slot],
        )
        tail = pltpu.make_async_copy(
            x_hbm.at[:, pl.ds((NCH - 1) * TCH + TAIL_A, TAIL_B)], scr_tail, sem_t
        )
        return full, part, tail

    def dma(i, slot):
        full, part, tail = descs(i, slot)

        @pl.when(i < NCH - 1)
        def _():
            full.start()

        @pl.when(i == NCH - 1)
        def _():
            part.start()
            tail.start()

    def dma_wait(i, slot):
        full, part, tail = descs(i, slot)

        @pl.when(i < NCH - 1)
        def _():
            full.wait()

        @pl.when(i == NCH - 1)
        def _():
            part.wait()
            tail.wait()

    @pl.when(g == 0)
    def _():
        for i in range(TNB - 1):
            dma(i, i)

    @pl.when(g + TNB - 1 < NCH)
    def _():
        dma(g + TNB - 1, lax.rem(g + TNB - 1, TNB))

    dma_wait(g, lax.rem(g, TNB))

    @pl.when(g == NCH - 1)
    def _():
        scr[lax.rem(g, TNB), :, TAIL_A:TAIL] = scr_tail[...]

    x = scr[lax.rem(g, TNB)]
    o_ref[:, 0:EMBED] = x[:, 0:HALF].T
    o_ref[:, EMBED : 2 * EMBED] = x[:, HALF:TCH].T


def _relayout_table(table_t):
    return pl.pallas_call(
        _tbl_relayout_body,
        grid=(NCH,),
        in_specs=[pl.BlockSpec(memory_space=pltpu.MemorySpace.HBM)],
        out_specs=pl.BlockSpec((HALF, 2 * EMBED), lambda g: (g, 0)),
        out_shape=jax.ShapeDtypeStruct((TBL2_ROWS, 2 * EMBED), jnp.float32),
        scratch_shapes=[
            pltpu.VMEM((TNB, EMBED, TCH), jnp.float32),
            pltpu.VMEM((EMBED, TAIL_B), jnp.float32),
            pltpu.SemaphoreType.DMA((TNB,)),
            pltpu.SemaphoreType.DMA,
        ],
    )(table_t)


NUM_CORES = 2
NUM_SUBCORES = 16
NW = NUM_CORES * NUM_SUBCORES
COLS_W = BATCH // NW
GL = 4
N_GROUPS = SEQ // GL
NBUF = 3


def _gather_body(idx_hbm, table_hbm, out_hbm, idx_v, rows_v, sem_g, sem_o):
    c = lax.axis_index("c")
    s = lax.axis_index("s")
    wid = s * NUM_CORES + c
    c0 = wid * COLS_W
    pltpu.sync_copy(idx_hbm.at[:, pl.ds(c0, COLS_W)], idx_v)

    def fire(g, slot):
        for j in range(GL):
            pltpu.make_async_copy(
                table_hbm.at[idx_v.at[g * GL + j]],
                rows_v.at[slot, j],
                sem_g.at[slot],
            ).start()

    def drain(g, slot):
        for j in range(GL):
            pltpu.make_async_copy(
                table_hbm.at[idx_v.at[g * GL + j]],
                rows_v.at[slot, j],
                sem_g.at[slot],
            ).wait()

    def store_desc(g, slot):
        return pltpu.make_async_copy(
            rows_v.at[slot],
            out_hbm.at[pl.ds(g * GL, GL), pl.ds(c0, COLS_W)],
            sem_o.at[slot],
        )

    fire(0, 0)
    fire(1, 1)

    def step(g, carry):
        slot = lax.rem(g, NBUF)
        drain(g, slot)
        store_desc(g, slot).start()
        slot_next = lax.rem(g + 2, NBUF)

        @pl.when(g >= 1)
        def _():
            store_desc(g - 1, slot_next).wait()

        @pl.when(g + 2 < N_GROUPS)
        def _():
            fire(g + 2, slot_next)

        return carry

    lax.fori_loop(0, N_GROUPS, step, 0)
    store_desc(N_GROUPS - 1, lax.rem(N_GROUPS - 1, NBUF)).wait()


def _embedding_gather(idx_p, table_lin):
    mesh = plsc.VectorSubcoreMesh(core_axis_name="c", subcore_axis_name="s")
    kern = functools.partial(
        pl.kernel,
        mesh=mesh,
        out_type=jax.ShapeDtypeStruct((SEQ, BATCH, EMBED), jnp.float32),
        scratch_types=[
            pltpu.VMEM((SEQ, COLS_W), jnp.int32),
            pltpu.VMEM((NBUF, GL, COLS_W, EMBED), jnp.float32),
            pltpu.SemaphoreType.DMA((NBUF,)),
            pltpu.SemaphoreType.DMA((NBUF,)),
        ],
        compiler_params=pltpu.CompilerParams(use_tc_tiling_on_sc=False),
    )(_gather_body)
    return kern(idx_p, table_lin)


HB = BATCH // 2


LPB = 2


def _out_relayout_body(x_ref, o_ref):
    for lp in range(LPB):
        x = x_ref[pl.ds(lp * HB, HB), :]
        o_ref[lp, :, 0:HB] = x[:, 0:EMBED].T
        o_ref[lp, :, HB:BATCH] = x[:, EMBED : 2 * EMBED].T


def _relayout_out(out2_rows):
    return pl.pallas_call(
        _out_relayout_body,
        grid=(SEQ // LPB,),
        in_specs=[pl.BlockSpec((LPB * HB, 2 * EMBED), lambda l: (l, 0))],
        out_specs=pl.BlockSpec((LPB, EMBED, BATCH), lambda l: (l, 0, 0)),
        out_shape=jax.ShapeDtypeStruct((SEQ, EMBED, BATCH), jnp.float32),
    )(out2_rows)


@jax.jit
def _run(input_sequence, word_embed_weight):
    tbl2 = _relayout_table(word_embed_weight.T)
    tbl_lin = tbl2.reshape(2 * TBL2_ROWS, EMBED)

    v = input_sequence.astype(jnp.int32)
    q = (v // TCH) * TCH + (v % HALF) * 2 + (v % TCH) // HALF
    qt = q.T
    idx_p = jnp.stack((qt[:, :HB], qt[:, HB:]), axis=2).reshape(SEQ, BATCH)

    out2 = _embedding_gather(idx_p, tbl_lin)
    out_t = _relayout_out(out2.reshape(SEQ * HB, 2 * EMBED))
    return out_t.transpose(2, 0, 1)


def kernel(input_sequence, word_embed_weight):
    return _run(input_sequence, word_embed_weight)


# speedup vs baseline: 2.1951x; 1.0846x over previous
import functools

import jax
import jax.numpy as jnp
from jax import lax
from jax.experimental import pallas as pl
from jax.experimental.pallas import tpu as pltpu
from jax.experimental.pallas import tpu_sc as plsc

VOCAB = 1000000
EMBED = 64
BATCH = 4096
SEQ = 200


TCH = 8192
NCH = (VOCAB + TCH - 1) // TCH
TAIL = VOCAB - (NCH - 1) * TCH
HALF = TCH // 2
TBL2_ROWS = NCH * HALF
TAIL_A = TAIL - (TAIL % 128)
TAIL_B = TAIL % 128
TNB = 4


def _tbl_relayout_body(x_hbm, o_ref, scr, scr_tail, sem, sem_t):
    g = pl.program_id(0)

    def descs(i, slot):
        full = pltpu.make_async_copy(
            x_hbm.at[:, pl.ds(i * TCH, TCH)], scr.at[slot], sem.at[slot]
        )
        part = pltpu.make_async_copy(
            x_hbm.at[:, pl.ds((NCH - 1) * TCH, TAIL_A)],
            scr.at[slot, :, pl.ds(0, TAIL_A)],
            sem.at[slot],
        )
        tail = pltpu.make_async_copy(
            x_hbm.at[:, pl.ds((NCH - 1) * TCH + TAIL_A, TAIL_B)], scr_tail, sem_t
        )
        return full, part, tail

    def dma(i, slot):
        full, part, tail = descs(i, slot)

        @pl.when(i < NCH - 1)
        def _():
            full.start()

        @pl.when(i == NCH - 1)
        def _():
            part.start()
            tail.start()

    def dma_wait(i, slot):
        full, part, tail = descs(i, slot)

        @pl.when(i < NCH - 1)
        def _():
            full.wait()

        @pl.when(i == NCH - 1)
        def _():
            part.wait()
            tail.wait()

    @pl.when(g == 0)
    def _():
        for i in range(TNB - 1):
            dma(i, i)

    @pl.when(g + TNB - 1 < NCH)
    def _():
        dma(g + TNB - 1, lax.rem(g + TNB - 1, TNB))

    dma_wait(g, lax.rem(g, TNB))

    @pl.when(g == NCH - 1)
    def _():
        scr[lax.rem(g, TNB), :, TAIL_A:TAIL] = scr_tail[...]

    x = scr[lax.rem(g, TNB)]
    o_ref[:, 0:EMBED] = x[:, 0:HALF].T
    o_ref[:, EMBED : 2 * EMBED] = x[:, HALF:TCH].T


def _relayout_table(table_t):
    return pl.pallas_call(
        _tbl_relayout_body,
        grid=(NCH,),
        in_specs=[pl.BlockSpec(memory_space=pltpu.MemorySpace.HBM)],
        out_specs=pl.BlockSpec((HALF, 2 * EMBED), lambda g: (g, 0)),
        out_shape=jax.ShapeDtypeStruct((TBL2_ROWS, 2 * EMBED), jnp.float32),
        scratch_shapes=[
            pltpu.VMEM((TNB, EMBED, TCH), jnp.float32),
            pltpu.VMEM((EMBED, TAIL_B), jnp.float32),
            pltpu.SemaphoreType.DMA((TNB,)),
            pltpu.SemaphoreType.DMA,
        ],
    )(table_t)


NUM_CORES = 2
NUM_SUBCORES = 16
NW = NUM_CORES * NUM_SUBCORES
COLS_W = BATCH // NW
GL = 4
N_GROUPS = SEQ // GL
NBUF = 3


def _gather_body(idx_hbm, table_hbm, out_hbm, idx_v, rows_v, sem_g, sem_o):
    c = lax.axis_index("c")
    s = lax.axis_index("s")
    wid = s * NUM_CORES + c
    c0 = wid * COLS_W
    pltpu.sync_copy(idx_hbm.at[:, pl.ds(c0, COLS_W)], idx_v)

    def fire(g, slot):
        for j in range(GL):
            pltpu.make_async_copy(
                table_hbm.at[idx_v.at[g * GL + j]],
                rows_v.at[slot, j],
                sem_g.at[slot],
            ).start()

    def drain(g, slot):
        for j in range(GL):
            pltpu.make_async_copy(
                table_hbm.at[idx_v.at[g * GL + j]],
                rows_v.at[slot, j],
                sem_g.at[slot],
            ).wait()

    def store_desc(g, slot):
        return pltpu.make_async_copy(
            rows_v.at[slot],
            out_hbm.at[pl.ds(g * GL, GL), pl.ds(c0, COLS_W)],
            sem_o.at[slot],
        )

    fire(0, 0)
    fire(1, 1)

    def step(g, carry):
        slot = lax.rem(g, NBUF)
        drain(g, slot)
        store_desc(g, slot).start()
        slot_next = lax.rem(g + 2, NBUF)

        @pl.when(g >= 1)
        def _():
            store_desc(g - 1, slot_next).wait()

        @pl.when(g + 2 < N_GROUPS)
        def _():
            fire(g + 2, slot_next)

        return carry

    lax.fori_loop(0, N_GROUPS, step, 0)
    store_desc(N_GROUPS - 1, lax.rem(N_GROUPS - 1, NBUF)).wait()


def _embedding_gather(idx_p, table_lin):
    mesh = plsc.VectorSubcoreMesh(core_axis_name="c", subcore_axis_name="s")
    kern = functools.partial(
        pl.kernel,
        mesh=mesh,
        out_type=jax.ShapeDtypeStruct((SEQ, BATCH, EMBED), jnp.float32),
        scratch_types=[
            pltpu.VMEM((SEQ, COLS_W), jnp.int32),
            pltpu.VMEM((NBUF, GL, COLS_W, EMBED), jnp.float32),
            pltpu.SemaphoreType.DMA((NBUF,)),
            pltpu.SemaphoreType.DMA((NBUF,)),
        ],
        compiler_params=pltpu.CompilerParams(use_tc_tiling_on_sc=False),
    )(_gather_body)
    return kern(idx_p, table_lin)


HB = BATCH // 2


LPB = 4


def _out_relayout_body(x_ref, o_ref):
    for lp in range(LPB):
        x = x_ref[pl.ds(lp * HB, HB), :]
        o_ref[lp, :, 0:HB] = x[:, 0:EMBED].T
        o_ref[lp, :, HB:BATCH] = x[:, EMBED : 2 * EMBED].T


def _relayout_out(out2_rows):
    return pl.pallas_call(
        _out_relayout_body,
        grid=(SEQ // LPB,),
        in_specs=[pl.BlockSpec((LPB * HB, 2 * EMBED), lambda l: (l, 0))],
        out_specs=pl.BlockSpec((LPB, EMBED, BATCH), lambda l: (l, 0, 0)),
        out_shape=jax.ShapeDtypeStruct((SEQ, EMBED, BATCH), jnp.float32),
    )(out2_rows)


@jax.jit
def _run(input_sequence, word_embed_weight):
    tbl2 = _relayout_table(word_embed_weight.T)
    tbl_lin = tbl2.reshape(2 * TBL2_ROWS, EMBED)

    v = input_sequence.astype(jnp.int32)
    q = (v // TCH) * TCH + (v % HALF) * 2 + (v % TCH) // HALF
    qt = q.T
    idx_p = jnp.stack((qt[:, :HB], qt[:, HB:]), axis=2).reshape(SEQ, BATCH)

    out2 = _embedding_gather(idx_p, tbl_lin)
    out_t = _relayout_out(out2.reshape(SEQ * HB, 2 * EMBED))
    return out_t.transpose(2, 0, 1)


def kernel(input_sequence, word_embed_weight):
    return _run(input_sequence, word_embed_weight)


# speedup vs baseline: 2.2575x; 1.0284x over previous
import functools

import jax
import jax.numpy as jnp
from jax import lax
from jax.experimental import pallas as pl
from jax.experimental.pallas import tpu as pltpu
from jax.experimental.pallas import tpu_sc as plsc

VOCAB = 1000000
EMBED = 64
BATCH = 4096
SEQ = 200


TCH = 16384
NCH = (VOCAB + TCH - 1) // TCH
TAIL = VOCAB - (NCH - 1) * TCH
HALF = TCH // 2
TBL2_ROWS = NCH * HALF
TAIL_A = TAIL - (TAIL % 128)
TAIL_B = TAIL % 128
TNB = 4


def _tbl_relayout_body(x_hbm, o_ref, scr, scr_tail, sem, sem_t):
    g = pl.program_id(0)

    def descs(i, slot):
        full = pltpu.make_async_copy(
            x_hbm.at[:, pl.ds(i * TCH, TCH)], scr.at[slot], sem.at[slot]
        )
        part = pltpu.make_async_copy(
            x_hbm.at[:, pl.ds((NCH - 1) * TCH, TAIL_A)],
            scr.at[slot, :, pl.ds(0, TAIL_A)],
            sem.at[slot],
        )
        tail = pltpu.make_async_copy(
            x_hbm.at[:, pl.ds((NCH - 1) * TCH + TAIL_A, TAIL_B)], scr_tail, sem_t
        )
        return full, part, tail

    def dma(i, slot):
        full, part, tail = descs(i, slot)

        @pl.when(i < NCH - 1)
        def _():
            full.start()

        @pl.when(i == NCH - 1)
        def _():
            part.start()
            tail.start()

    def dma_wait(i, slot):
        full, part, tail = descs(i, slot)

        @pl.when(i < NCH - 1)
        def _():
            full.wait()

        @pl.when(i == NCH - 1)
        def _():
            part.wait()
            tail.wait()

    @pl.when(g == 0)
    def _():
        for i in range(TNB - 1):
            dma(i, i)

    @pl.when(g + TNB - 1 < NCH)
    def _():
        dma(g + TNB - 1, lax.rem(g + TNB - 1, TNB))

    dma_wait(g, lax.rem(g, TNB))

    @pl.when(g == NCH - 1)
    def _():
        scr[lax.rem(g, TNB), :, TAIL_A:TAIL] = scr_tail[...]

    x = scr[lax.rem(g, TNB)]
    o_ref[:, 0:EMBED] = x[:, 0:HALF].T
    o_ref[:, EMBED : 2 * EMBED] = x[:, HALF:TCH].T


def _relayout_table(table_t):
    return pl.pallas_call(
        _tbl_relayout_body,
        grid=(NCH,),
        in_specs=[pl.BlockSpec(memory_space=pltpu.MemorySpace.HBM)],
        out_specs=pl.BlockSpec((HALF, 2 * EMBED), lambda g: (g, 0)),
        out_shape=jax.ShapeDtypeStruct((TBL2_ROWS, 2 * EMBED), jnp.float32),
        scratch_shapes=[
            pltpu.VMEM((TNB, EMBED, TCH), jnp.float32),
            pltpu.VMEM((EMBED, TAIL_B), jnp.float32),
            pltpu.SemaphoreType.DMA((TNB,)),
            pltpu.SemaphoreType.DMA,
        ],
    )(table_t)


NUM_CORES = 2
NUM_SUBCORES = 16
NW = NUM_CORES * NUM_SUBCORES
COLS_W = BATCH // NW
GL = 4
N_GROUPS = SEQ // GL
NBUF = 3


def _gather_body(idx_hbm, table_hbm, out_hbm, idx_v, rows_v, sem_g, sem_o):
    c = lax.axis_index("c")
    s = lax.axis_index("s")
    wid = s * NUM_CORES + c
    c0 = wid * COLS_W
    pltpu.sync_copy(idx_hbm.at[:, pl.ds(c0, COLS_W)], idx_v)

    def fire(g, slot):
        for j in range(GL):
            pltpu.make_async_copy(
                table_hbm.at[idx_v.at[g * GL + j]],
                rows_v.at[slot, j],
                sem_g.at[slot],
            ).start()

    def drain(g, slot):
        for j in range(GL):
            pltpu.make_async_copy(
                table_hbm.at[idx_v.at[g * GL + j]],
                rows_v.at[slot, j],
                sem_g.at[slot],
            ).wait()

    def store_desc(g, slot):
        return pltpu.make_async_copy(
            rows_v.at[slot],
            out_hbm.at[pl.ds(g * GL, GL), pl.ds(c0, COLS_W)],
            sem_o.at[slot],
        )

    fire(0, 0)
    fire(1, 1)

    def step(g, carry):
        slot = lax.rem(g, NBUF)
        drain(g, slot)
        store_desc(g, slot).start()
        slot_next = lax.rem(g + 2, NBUF)

        @pl.when(g >= 1)
        def _():
            store_desc(g - 1, slot_next).wait()

        @pl.when(g + 2 < N_GROUPS)
        def _():
            fire(g + 2, slot_next)

        return carry

    lax.fori_loop(0, N_GROUPS, step, 0)
    store_desc(N_GROUPS - 1, lax.rem(N_GROUPS - 1, NBUF)).wait()


def _embedding_gather(idx_p, table_lin):
    mesh = plsc.VectorSubcoreMesh(core_axis_name="c", subcore_axis_name="s")
    kern = functools.partial(
        pl.kernel,
        mesh=mesh,
        out_type=jax.ShapeDtypeStruct((SEQ, BATCH, EMBED), jnp.float32),
        scratch_types=[
            pltpu.VMEM((SEQ, COLS_W), jnp.int32),
            pltpu.VMEM((NBUF, GL, COLS_W, EMBED), jnp.float32),
            pltpu.SemaphoreType.DMA((NBUF,)),
            pltpu.SemaphoreType.DMA((NBUF,)),
        ],
        compiler_params=pltpu.CompilerParams(use_tc_tiling_on_sc=False),
    )(_gather_body)
    return kern(idx_p, table_lin)


HB = BATCH // 2


LPB = 8


def _out_relayout_body(x_ref, o_ref):
    for lp in range(LPB):
        x = x_ref[pl.ds(lp * HB, HB), :]
        o_ref[lp, :, 0:HB] = x[:, 0:EMBED].T
        o_ref[lp, :, HB:BATCH] = x[:, EMBED : 2 * EMBED].T


def _relayout_out(out2_rows):
    return pl.pallas_call(
        _out_relayout_body,
        grid=(SEQ // LPB,),
        in_specs=[pl.BlockSpec((LPB * HB, 2 * EMBED), lambda l: (l, 0))],
        out_specs=pl.BlockSpec((LPB, EMBED, BATCH), lambda l: (l, 0, 0)),
        out_shape=jax.ShapeDtypeStruct((SEQ, EMBED, BATCH), jnp.float32),
    )(out2_rows)


@jax.jit
def _run(input_sequence, word_embed_weight):
    tbl2 = _relayout_table(word_embed_weight.T)
    tbl_lin = tbl2.reshape(2 * TBL2_ROWS, EMBED)

    v = input_sequence.astype(jnp.int32)
    q = (v // TCH) * TCH + (v % HALF) * 2 + (v % TCH) // HALF
    qt = q.T
    idx_p = jnp.stack((qt[:, :HB], qt[:, HB:]), axis=2).reshape(SEQ, BATCH)

    out2 = _embedding_gather(idx_p, tbl_lin)
    out_t = _relayout_out(out2.reshape(SEQ * HB, 2 * EMBED))
    return out_t.transpose(2, 0, 1)


def kernel(input_sequence, word_embed_weight):
    return _run(input_sequence, word_embed_weight)
